# baseline (speedup 1.0000x reference)
import functools

import jax
import jax.numpy as jnp
from jax import lax
from jax.experimental import pallas as pl
from jax.experimental.pallas import tpu as pltpu
from jax.experimental.pallas import tpu_sc as plsc

B = 50
M = 200
D = 128
N = B * M
E = 320000
K = M // 2
NCLS = 11

NC = 2
NS = 16
HALF = D // NC
ROWS_PER_TILE = N // NS
CHUNK = 128
NBUF = 3
CHUNKS_PER_TILE = 162
E_PAD = NS * CHUNKS_PER_TILE * CHUNK
TRASH = N
N_SH = N + 16


def _segsum_plus_x(xs, srcg, dstg):
    mesh = plsc.VectorSubcoreMesh(core_axis_name="c", subcore_axis_name="s")

    @functools.partial(
        pl.kernel,
        mesh=mesh,
        out_type=jax.ShapeDtypeStruct((N, D), jnp.float32),
        compiler_params=pltpu.CompilerParams(use_tc_tiling_on_sc=False),
        scratch_types=[
            pltpu.VMEM((CHUNKS_PER_TILE, CHUNK), jnp.int32),
            pltpu.VMEM((CHUNKS_PER_TILE, CHUNK), jnp.int32),
            pltpu.VMEM((NBUF, CHUNK, HALF), jnp.float32),
            pltpu.VMEM((NBUF, CHUNK, HALF), jnp.float32),
            pltpu.VMEM_SHARED((N_SH, HALF), jnp.float32),
            pltpu.SemaphoreType.DMA((NBUF,)),
            pltpu.SemaphoreType.DMA((NBUF,)),
            pltpu.SemaphoreType.DMA((NBUF,)),
            pltpu.SemaphoreType.DMA((NBUF,)),
        ],
    )
    def seg_kernel(xs_hbm, src_hbm, dst_hbm, out_hbm, src_v, dst_v,
                   rows0, rows1, agg_sh, gs0, gs1, ss0, ss1):
        c = lax.axis_index("c")
        s = lax.axis_index("s")
        col0 = c * HALF
        row0 = s * ROWS_PER_TILE
        xc = xs_hbm.at[c]
        pltpu.sync_copy(xc.at[pl.ds(row0, ROWS_PER_TILE), :],
                        agg_sh.at[pl.ds(row0, ROWS_PER_TILE), :])
        pltpu.sync_copy(src_hbm.at[s], src_v)
        pltpu.sync_copy(dst_hbm.at[s], dst_v)
        plsc.subcore_barrier()

        rows = (rows0, rows1)
        gsem = (gs0, gs1)
        ssem = (ss0, ss1)

        def start_gather(j, p, b):
            pltpu.async_copy(xc.at[src_v.at[j]], rows[p].at[b], gsem[p].at[b])

        def wait_gather(j, p, b):
            pltpu.make_async_copy(xc.at[src_v.at[j]], rows[p].at[b],
                                  gsem[p].at[b]).wait()

        def start_scatter(j, p, b):
            pltpu.async_copy(rows[p].at[b], agg_sh.at[dst_v.at[j]],
                             ssem[p].at[b], add=True)

        def wait_scatter(j, p, b):
            pltpu.make_async_copy(rows[p].at[b], agg_sh.at[dst_v.at[j]],
                                  ssem[p].at[b]).wait()

        n_dbl = CHUNKS_PER_TILE // (2 * NBUF)
        for p in (0, 1):
            for b in range(NBUF):
                start_gather(p * NBUF + b, p, b)

        def dbl(t, carry):
            base = t * 2 * NBUF
            for p in (0, 1):
                for b in range(NBUF):
                    j = base + p * NBUF + b
                    wait_gather(j, p, b)
                    start_scatter(j, p, b)

                @pl.when(t < n_dbl - 1)
                def _():
                    for b in range(NBUF):
                        j = base + p * NBUF + b
                        wait_scatter(j, p, b)
                        start_gather(j + 2 * NBUF, p, b)
            return carry

        lax.fori_loop(0, n_dbl, dbl, 0)
        last = (n_dbl - 1) * 2 * NBUF
        for p in (0, 1):
            for b in range(NBUF):
                wait_scatter(last + p * NBUF + b, p, b)
        plsc.subcore_barrier()
        pltpu.sync_copy(agg_sh.at[pl.ds(row0, ROWS_PER_TILE), :],
                        out_hbm.at[pl.ds(row0, ROWS_PER_TILE), pl.ds(col0, HALF)])

    return seg_kernel(xs, srcg, dstg)


def _dense_relu(h, W, b):
    blk = 2000

    def body(h_ref, w_ref, b_ref, o1_ref, o2_ref):
        r = jnp.maximum(
            jnp.dot(h_ref[...], w_ref[...],
                    preferred_element_type=jnp.float32) + b_ref[...], 0.0)
        o1_ref[...] = r
        o2_ref[0, :, :] = r[:, :HALF]
        o2_ref[1, :, :] = r[:, HALF:]

    return pl.pallas_call(
        body,
        grid=(N // blk,),
        in_specs=[pl.BlockSpec((blk, D), lambda i: (i, 0)),
                  pl.BlockSpec((D, D), lambda i: (0, 0)),
                  pl.BlockSpec((1, D), lambda i: (0, 0))],
        out_specs=[pl.BlockSpec((blk, D), lambda i: (i, 0)),
                   pl.BlockSpec((2, blk, HALF), lambda i: (0, i, 0))],
        out_shape=[jax.ShapeDtypeStruct((N, D), jnp.float32),
                   jax.ShapeDtypeStruct((2, N, HALF), jnp.float32)],
    )(h, W, b.reshape(1, D))


def _head(h3, x2, wscore, wcls_pad, bcls_pad):
    def body(h_ref, x_ref, ws_ref, wc_ref, bc_ref, o_ref):
        h = h_ref[...]
        s = jnp.tanh(lax.dot_general(h, ws_ref[...], (((1,), (0,)), ((), ())),
                                     preferred_element_type=jnp.float32))
        ii = lax.broadcasted_iota(jnp.int32, (M, M), 0)
        jj = lax.broadcasted_iota(jnp.int32, (M, M), 1)
        eye = (ii == jj).astype(jnp.float32)
        s_row = lax.dot_general(s, eye, (((0,), (0,)), ((), ())),
                                preferred_element_type=jnp.float32)
        s_col_b = lax.broadcast_in_dim(s, (M, M), (0, 1))
        s_row_b = lax.broadcast_in_dim(s_row, (M, M), (0, 1))
        beats = (s_row_b > s_col_b) | ((s_row_b == s_col_b) & (jj < ii))
        rank = jnp.sum(beats.astype(jnp.float32), axis=1, keepdims=True)
        w = jnp.where(rank < float(K), s, 0.0) * (1.0 / K)
        pooled = lax.dot_general(w, x_ref[...], (((0,), (0,)), ((), ())),
                                 preferred_element_type=jnp.float32)
        emb = jnp.maximum(pooled, 0.0)
        logits = jnp.dot(emb, wc_ref[...],
                         preferred_element_type=jnp.float32) + bc_ref[...]
        o_ref[...] = lax.broadcast_in_dim(logits, (8, D), (0, 1))

    out = pl.pallas_call(
        body,
        grid=(B,),
        in_specs=[pl.BlockSpec((M, D), lambda i: (i, 0)),
                  pl.BlockSpec((M, D), lambda i: (i, 0)),
                  pl.BlockSpec((D, 1), lambda i: (0, 0)),
                  pl.BlockSpec((D, D), lambda i: (0, 0)),
                  pl.BlockSpec((1, D), lambda i: (0, 0))],
        out_specs=pl.BlockSpec((8, D), lambda i: (i, 0)),
        out_shape=jax.ShapeDtypeStruct((B * 8, D), jnp.float32),
    )(h3, x2, wscore.reshape(D, 1), wcls_pad, bcls_pad)
    return out[::8]


def kernel(node_feats, collated_edge_index, W1, b1, W2, b2, wscore, Wcls, bcls):
    x = node_feats.reshape(N, D).astype(jnp.float32)
    src = collated_edge_index[0].astype(jnp.int32)
    dst = collated_edge_index[1].astype(jnp.int32)
    pad = E_PAD - E
    srcg = jnp.concatenate([src, jnp.zeros((pad,), jnp.int32)]).reshape(
        NS, CHUNKS_PER_TILE, CHUNK)
    dstg = jnp.concatenate([dst, jnp.full((pad,), TRASH, jnp.int32)]).reshape(
        NS, CHUNKS_PER_TILE, CHUNK)

    xs0 = jnp.stack([x[:, :HALF], x[:, HALF:]])
    h1 = _segsum_plus_x(xs0, srcg, dstg)
    _, x1s = _dense_relu(h1, W1, b1)
    h2 = _segsum_plus_x(x1s, srcg, dstg)
    x2, x2s = _dense_relu(h2, W2, b2)
    h3 = _segsum_plus_x(x2s, srcg, dstg)

    wcls_pad = jnp.zeros((D, D), jnp.float32).at[:, :NCLS].set(Wcls)
    bcls_pad = jnp.zeros((1, D), jnp.float32).at[0, :NCLS].set(bcls)
    out = _head(h3, x2, wscore, wcls_pad, bcls_pad)
    return out[:, :NCLS]


# speedup vs baseline: 2.2976x; 2.2976x over previous
import functools

import jax
import jax.numpy as jnp
from jax import lax
from jax.experimental import pallas as pl
from jax.experimental.pallas import tpu as pltpu
from jax.experimental.pallas import tpu_sc as plsc

B = 50
M = 200
D = 128
N = B * M
E = 320000
K = M // 2
NCLS = 11

NC = 2
NS = 16
HALF = D // NC
ROWS_PER_TILE = N // NS
CHUNK = 96
NBUF = 2
CHUNKS_PER_TILE = 212
PK = CHUNK // 2
E_PAD = NS * CHUNKS_PER_TILE * CHUNK
TRASH = N
N_SH = N + 16


def _pack_idx(idx_flat):
    g = idx_flat.reshape(NS, CHUNKS_PER_TILE, CHUNK // 32, 2, 16)
    lo = g[:, :, :, 0, :]
    hi = g[:, :, :, 1, :]
    return (lo | (hi << 16)).reshape(NS, CHUNKS_PER_TILE, PK)


def _segsum_plus_x(x, srcpk, dstpk):
    mesh = plsc.VectorSubcoreMesh(core_axis_name="c", subcore_axis_name="s")

    @functools.partial(
        pl.kernel,
        mesh=mesh,
        out_type=jax.ShapeDtypeStruct((N, D), jnp.float32),
        compiler_params=pltpu.CompilerParams(use_tc_tiling_on_sc=False),
        scratch_types=[
            pltpu.VMEM((CHUNKS_PER_TILE, PK), jnp.int32),
            pltpu.VMEM((CHUNKS_PER_TILE, PK), jnp.int32),
            pltpu.VMEM((2, NBUF, CHUNK), jnp.int32),
            pltpu.VMEM((2, NBUF, CHUNK), jnp.int32),
            pltpu.VMEM((NBUF, CHUNK, HALF), jnp.float32),
            pltpu.VMEM((NBUF, CHUNK, HALF), jnp.float32),
            pltpu.VMEM_SHARED((N_SH, HALF), jnp.float32),
            pltpu.VMEM_SHARED((N_SH, HALF), jnp.float32),
            pltpu.SemaphoreType.DMA((NBUF,)),
            pltpu.SemaphoreType.DMA((NBUF,)),
            pltpu.SemaphoreType.DMA((NBUF,)),
            pltpu.SemaphoreType.DMA((NBUF,)),
        ],
    )
    def seg_kernel(x_hbm, src_hbm, dst_hbm, out_hbm, spk_v, dpk_v,
                   sidx, didx, rows0, rows1, x_sh, agg_sh,
                   gs0, gs1, ss0, ss1):
        c = lax.axis_index("c")
        s = lax.axis_index("s")
        col0 = c * HALF
        row0 = s * ROWS_PER_TILE
        pltpu.sync_copy(x_hbm.at[pl.ds(row0, ROWS_PER_TILE), pl.ds(col0, HALF)],
                        x_sh.at[pl.ds(row0, ROWS_PER_TILE), :])
        pltpu.sync_copy(x_hbm.at[pl.ds(row0, ROWS_PER_TILE), pl.ds(col0, HALF)],
                        agg_sh.at[pl.ds(row0, ROWS_PER_TILE), :])
        pltpu.sync_copy(src_hbm.at[s], spk_v)
        pltpu.sync_copy(dst_hbm.at[s], dpk_v)
        plsc.subcore_barrier()

        rows = (rows0, rows1)
        gsem = (gs0, gs1)
        ssem = (ss0, ss1)

        def decode(j, p, b):
            for pk_v, out in ((spk_v, sidx), (dpk_v, didx)):
                for m in range(CHUNK // 32):
                    v = pk_v[j, pl.ds(m * 16, 16)]
                    out[p, b, pl.ds(m * 32, 16)] = v & 0xFFFF
                    out[p, b, pl.ds(m * 32 + 16, 16)] = (
                        lax.shift_right_logical(v, 16))

        def start_gather(p, b):
            pltpu.async_copy(x_sh.at[sidx.at[p, b]], rows[p].at[b],
                             gsem[p].at[b])

        def wait_gather(p, b):
            pltpu.make_async_copy(x_sh.at[sidx.at[p, b]], rows[p].at[b],
                                  gsem[p].at[b]).wait()

        def start_scatter(p, b):
            pltpu.async_copy(rows[p].at[b], agg_sh.at[didx.at[p, b]],
                             ssem[p].at[b], add=True)

        def wait_scatter(p, b):
            pltpu.make_async_copy(rows[p].at[b], agg_sh.at[didx.at[p, b]],
                                  ssem[p].at[b]).wait()

        n_dbl = CHUNKS_PER_TILE // (2 * NBUF)
        for p in (0, 1):
            for b in range(NBUF):
                decode(p * NBUF + b, p, b)
                start_gather(p, b)

        def dbl(t, carry):
            base = t * 2 * NBUF
            for p in (0, 1):
                for b in range(NBUF):
                    wait_gather(p, b)
                    start_scatter(p, b)

                @pl.when(t < n_dbl - 1)
                def _():
                    for b in range(NBUF):
                        wait_scatter(p, b)
                        decode(base + p * NBUF + b + 2 * NBUF, p, b)
                        start_gather(p, b)
            return carry

        lax.fori_loop(0, n_dbl, dbl, 0)
        for p in (0, 1):
            for b in range(NBUF):
                wait_scatter(p, b)
        plsc.subcore_barrier()
        pltpu.sync_copy(agg_sh.at[pl.ds(row0, ROWS_PER_TILE), :],
                        out_hbm.at[pl.ds(row0, ROWS_PER_TILE), pl.ds(col0, HALF)])

    return seg_kernel(x, srcpk, dstpk)


def _dense_relu(h, W, b):
    blk = 2000

    def body(h_ref, w_ref, b_ref, o_ref):
        o_ref[...] = jnp.maximum(
            jnp.dot(h_ref[...], w_ref[...],
                    preferred_element_type=jnp.float32) + b_ref[...], 0.0)

    return pl.pallas_call(
        body,
        grid=(N // blk,),
        in_specs=[pl.BlockSpec((blk, D), lambda i: (i, 0)),
                  pl.BlockSpec((D, D), lambda i: (0, 0)),
                  pl.BlockSpec((1, D), lambda i: (0, 0))],
        out_specs=pl.BlockSpec((blk, D), lambda i: (i, 0)),
        out_shape=jax.ShapeDtypeStruct((N, D), jnp.float32),
    )(h, W, b.reshape(1, D))


def _head(h3, x2, wscore, wcls_pad, bcls_pad):
    def body(h_ref, x_ref, ws_ref, wc_ref, bc_ref, o_ref):
        h = h_ref[...]
        s = jnp.tanh(lax.dot_general(h, ws_ref[...], (((1,), (0,)), ((), ())),
                                     preferred_element_type=jnp.float32))
        ii = lax.broadcasted_iota(jnp.int32, (M, M), 0)
        jj = lax.broadcasted_iota(jnp.int32, (M, M), 1)
        eye = (ii == jj).astype(jnp.float32)
        s_row = lax.dot_general(s, eye, (((0,), (0,)), ((), ())),
                                preferred_element_type=jnp.float32)
        s_col_b = lax.broadcast_in_dim(s, (M, M), (0, 1))
        s_row_b = lax.broadcast_in_dim(s_row, (M, M), (0, 1))
        beats = (s_row_b > s_col_b) | ((s_row_b == s_col_b) & (jj < ii))
        rank = jnp.sum(beats.astype(jnp.float32), axis=1, keepdims=True)
        w = jnp.where(rank < float(K), s, 0.0) * (1.0 / K)
        pooled = lax.dot_general(w, x_ref[...], (((0,), (0,)), ((), ())),
                                 preferred_element_type=jnp.float32)
        emb = jnp.maximum(pooled, 0.0)
        logits = jnp.dot(emb, wc_ref[...],
                         preferred_element_type=jnp.float32) + bc_ref[...]
        o_ref[...] = lax.broadcast_in_dim(logits, (8, D), (0, 1))

    out = pl.pallas_call(
        body,
        grid=(B,),
        in_specs=[pl.BlockSpec((M, D), lambda i: (i, 0)),
                  pl.BlockSpec((M, D), lambda i: (i, 0)),
                  pl.BlockSpec((D, 1), lambda i: (0, 0)),
                  pl.BlockSpec((D, D), lambda i: (0, 0)),
                  pl.BlockSpec((1, D), lambda i: (0, 0))],
        out_specs=pl.BlockSpec((8, D), lambda i: (i, 0)),
        out_shape=jax.ShapeDtypeStruct((B * 8, D), jnp.float32),
    )(h3, x2, wscore.reshape(D, 1), wcls_pad, bcls_pad)
    return out[::8]


def kernel(node_feats, collated_edge_index, W1, b1, W2, b2, wscore, Wcls, bcls):
    x = node_feats.reshape(N, D).astype(jnp.float32)
    src = collated_edge_index[0].astype(jnp.int32)
    dst = collated_edge_index[1].astype(jnp.int32)
    pad = E_PAD - E
    srcpk = _pack_idx(jnp.concatenate([src, jnp.zeros((pad,), jnp.int32)]))
    dstpk = _pack_idx(jnp.concatenate([dst, jnp.full((pad,), TRASH, jnp.int32)]))

    h1 = _segsum_plus_x(x, srcpk, dstpk)
    x1 = _dense_relu(h1, W1, b1)
    h2 = _segsum_plus_x(x1, srcpk, dstpk)
    x2 = _dense_relu(h2, W2, b2)
    h3 = _segsum_plus_x(x2, srcpk, dstpk)

    wcls_pad = jnp.zeros((D, D), jnp.float32).at[:, :NCLS].set(Wcls)
    bcls_pad = jnp.zeros((1, D), jnp.float32).at[0, :NCLS].set(bcls)
    out = _head(h3, x2, wscore, wcls_pad, bcls_pad)
    return out[:, :NCLS]


# speedup vs baseline: 2.4020x; 1.0455x over previous
import functools

import jax
import jax.numpy as jnp
from jax import lax
from jax.experimental import pallas as pl
from jax.experimental.pallas import tpu as pltpu
from jax.experimental.pallas import tpu_sc as plsc

B = 50
M = 200
D = 128
N = B * M
E = 320000
K = M // 2
NCLS = 11

NC = 2
NS = 16
HALF = D // NC
ROWS_PER_TILE = N // NS
CHUNK = 64
NBUF = 3
CHUNKS_PER_TILE = 318
PK = CHUNK // 2
E_PAD = NS * CHUNKS_PER_TILE * CHUNK
TRASH = N
N_SH = N + 16


def _pack_idx(idx_flat):
    g = idx_flat.reshape(NS, CHUNKS_PER_TILE, CHUNK // 32, 2, 16)
    lo = g[:, :, :, 0, :]
    hi = g[:, :, :, 1, :]
    return (lo | (hi << 16)).reshape(NS, CHUNKS_PER_TILE, PK)


def _segsum_plus_x(x, srcpk, dstpk):
    mesh = plsc.VectorSubcoreMesh(core_axis_name="c", subcore_axis_name="s")

    @functools.partial(
        pl.kernel,
        mesh=mesh,
        out_type=jax.ShapeDtypeStruct((N, D), jnp.float32),
        compiler_params=pltpu.CompilerParams(use_tc_tiling_on_sc=False),
        scratch_types=[
            pltpu.VMEM((CHUNKS_PER_TILE, PK), jnp.int32),
            pltpu.VMEM((CHUNKS_PER_TILE, PK), jnp.int32),
            pltpu.VMEM((2, NBUF, CHUNK), jnp.int32),
            pltpu.VMEM((2, NBUF, CHUNK), jnp.int32),
            pltpu.VMEM((NBUF, CHUNK, HALF), jnp.float32),
            pltpu.VMEM((NBUF, CHUNK, HALF), jnp.float32),
            pltpu.VMEM_SHARED((N_SH, HALF), jnp.float32),
            pltpu.VMEM_SHARED((N_SH, HALF), jnp.float32),
            pltpu.SemaphoreType.DMA((NBUF,)),
            pltpu.SemaphoreType.DMA((NBUF,)),
            pltpu.SemaphoreType.DMA((NBUF,)),
            pltpu.SemaphoreType.DMA((NBUF,)),
        ],
    )
    def seg_kernel(x_hbm, src_hbm, dst_hbm, out_hbm, spk_v, dpk_v,
                   sidx, didx, rows0, rows1, x_sh, agg_sh,
                   gs0, gs1, ss0, ss1):
        c = lax.axis_index("c")
        s = lax.axis_index("s")
        col0 = c * HALF
        row0 = s * ROWS_PER_TILE
        pltpu.sync_copy(x_hbm.at[pl.ds(row0, ROWS_PER_TILE), pl.ds(col0, HALF)],
                        x_sh.at[pl.ds(row0, ROWS_PER_TILE), :])
        pltpu.sync_copy(x_hbm.at[pl.ds(row0, ROWS_PER_TILE), pl.ds(col0, HALF)],
                        agg_sh.at[pl.ds(row0, ROWS_PER_TILE), :])
        pltpu.sync_copy(src_hbm.at[s], spk_v)
        pltpu.sync_copy(dst_hbm.at[s], dpk_v)
        plsc.subcore_barrier()

        rows = (rows0, rows1)
        gsem = (gs0, gs1)
        ssem = (ss0, ss1)

        def decode(j, p, b):
            for pk_v, out in ((spk_v, sidx), (dpk_v, didx)):
                for m in range(CHUNK // 32):
                    v = pk_v[j, pl.ds(m * 16, 16)]
                    out[p, b, pl.ds(m * 32, 16)] = v & 0xFFFF
                    out[p, b, pl.ds(m * 32 + 16, 16)] = (
                        lax.shift_right_logical(v, 16))

        def start_gather(p, b):
            pltpu.async_copy(x_sh.at[sidx.at[p, b]], rows[p].at[b],
                             gsem[p].at[b])

        def wait_gather(p, b):
            pltpu.make_async_copy(x_sh.at[sidx.at[p, b]], rows[p].at[b],
                                  gsem[p].at[b]).wait()

        def start_scatter(p, b):
            pltpu.async_copy(rows[p].at[b], agg_sh.at[didx.at[p, b]],
                             ssem[p].at[b], add=True)

        def wait_scatter(p, b):
            pltpu.make_async_copy(rows[p].at[b], agg_sh.at[didx.at[p, b]],
                                  ssem[p].at[b]).wait()

        n_dbl = CHUNKS_PER_TILE // (2 * NBUF)
        for p in (0, 1):
            for b in range(NBUF):
                decode(p * NBUF + b, p, b)
                start_gather(p, b)

        def dbl(t, carry):
            base = t * 2 * NBUF
            for p in (0, 1):
                for b in range(NBUF):
                    wait_gather(p, b)
                    start_scatter(p, b)

                @pl.when(t < n_dbl - 1)
                def _():
                    for b in range(NBUF):
                        wait_scatter(p, b)
                        decode(base + p * NBUF + b + 2 * NBUF, p, b)
                        start_gather(p, b)
            return carry

        lax.fori_loop(0, n_dbl, dbl, 0)
        for p in (0, 1):
            for b in range(NBUF):
                wait_scatter(p, b)
        plsc.subcore_barrier()
        pltpu.sync_copy(agg_sh.at[pl.ds(row0, ROWS_PER_TILE), :],
                        out_hbm.at[pl.ds(row0, ROWS_PER_TILE), pl.ds(col0, HALF)])

    return seg_kernel(x, srcpk, dstpk)


def _dense_relu(h, W, b):
    blk = 2000

    def body(h_ref, w_ref, b_ref, o_ref):
        o_ref[...] = jnp.maximum(
            jnp.dot(h_ref[...], w_ref[...],
                    preferred_element_type=jnp.float32) + b_ref[...], 0.0)

    return pl.pallas_call(
        body,
        grid=(N // blk,),
        in_specs=[pl.BlockSpec((blk, D), lambda i: (i, 0)),
                  pl.BlockSpec((D, D), lambda i: (0, 0)),
                  pl.BlockSpec((1, D), lambda i: (0, 0))],
        out_specs=pl.BlockSpec((blk, D), lambda i: (i, 0)),
        out_shape=jax.ShapeDtypeStruct((N, D), jnp.float32),
    )(h, W, b.reshape(1, D))


def _head(h3, x2, wscore, wcls_pad, bcls_pad):
    def body(h_ref, x_ref, ws_ref, wc_ref, bc_ref, o_ref):
        h = h_ref[...]
        s = jnp.tanh(lax.dot_general(h, ws_ref[...], (((1,), (0,)), ((), ())),
                                     preferred_element_type=jnp.float32))
        ii = lax.broadcasted_iota(jnp.int32, (M, M), 0)
        jj = lax.broadcasted_iota(jnp.int32, (M, M), 1)
        eye = (ii == jj).astype(jnp.float32)
        s_row = lax.dot_general(s, eye, (((0,), (0,)), ((), ())),
                                preferred_element_type=jnp.float32)
        s_col_b = lax.broadcast_in_dim(s, (M, M), (0, 1))
        s_row_b = lax.broadcast_in_dim(s_row, (M, M), (0, 1))
        beats = (s_row_b > s_col_b) | ((s_row_b == s_col_b) & (jj < ii))
        rank = jnp.sum(beats.astype(jnp.float32), axis=1, keepdims=True)
        w = jnp.where(rank < float(K), s, 0.0) * (1.0 / K)
        pooled = lax.dot_general(w, x_ref[...], (((0,), (0,)), ((), ())),
                                 preferred_element_type=jnp.float32)
        emb = jnp.maximum(pooled, 0.0)
        logits = jnp.dot(emb, wc_ref[...],
                         preferred_element_type=jnp.float32) + bc_ref[...]
        o_ref[...] = lax.broadcast_in_dim(logits, (8, D), (0, 1))

    out = pl.pallas_call(
        body,
        grid=(B,),
        in_specs=[pl.BlockSpec((M, D), lambda i: (i, 0)),
                  pl.BlockSpec((M, D), lambda i: (i, 0)),
                  pl.BlockSpec((D, 1), lambda i: (0, 0)),
                  pl.BlockSpec((D, D), lambda i: (0, 0)),
                  pl.BlockSpec((1, D), lambda i: (0, 0))],
        out_specs=pl.BlockSpec((8, D), lambda i: (i, 0)),
        out_shape=jax.ShapeDtypeStruct((B * 8, D), jnp.float32),
    )(h3, x2, wscore.reshape(D, 1), wcls_pad, bcls_pad)
    return out[::8]


def kernel(node_feats, collated_edge_index, W1, b1, W2, b2, wscore, Wcls, bcls):
    x = node_feats.reshape(N, D).astype(jnp.float32)
    src = collated_edge_index[0].astype(jnp.int32)
    dst = collated_edge_index[1].astype(jnp.int32)
    pad = E_PAD - E
    srcpk = _pack_idx(jnp.concatenate([src, jnp.zeros((pad,), jnp.int32)]))
    dstpk = _pack_idx(jnp.concatenate([dst, jnp.full((pad,), TRASH, jnp.int32)]))

    h1 = _segsum_plus_x(x, srcpk, dstpk)
    x1 = _dense_relu(h1, W1, b1)
    h2 = _segsum_plus_x(x1, srcpk, dstpk)
    x2 = _dense_relu(h2, W2, b2)
    h3 = _segsum_plus_x(x2, srcpk, dstpk)

    wcls_pad = jnp.zeros((D, D), jnp.float32).at[:, :NCLS].set(Wcls)
    bcls_pad = jnp.zeros((1, D), jnp.float32).at[0, :NCLS].set(bcls)
    out = _head(h3, x2, wscore, wcls_pad, bcls_pad)
    return out[:, :NCLS]


# speedup vs baseline: 2.9309x; 1.2202x over previous
import functools

import jax
import jax.numpy as jnp
from jax import lax
from jax.experimental import pallas as pl
from jax.experimental.pallas import tpu as pltpu
from jax.experimental.pallas import tpu_sc as plsc

B = 50
M = 200
D = 128
N = B * M
E = 320000
K = M // 2
NCLS = 11

NC = 2
NS = 16
HALF = D // NC
ROWS_PER_TILE = N // NS
CHUNK = 64
NBUF = 3
CHUNKS_PER_TILE = 318
PK = CHUNK // 2
E_PAD = NS * CHUNKS_PER_TILE * CHUNK
TRASH = N
N_SH = N + 16


def _pack_idx(idx, pad_value):
    half = E_PAD // 2
    lo = idx[:half]
    hi = jnp.pad(idx[half:], (0, E_PAD - E), constant_values=pad_value)
    return (lo | (hi << 16)).reshape(NS, CHUNKS_PER_TILE, PK)


def _segsum_plus_x(x, srcpk, dstpk):
    mesh = plsc.VectorSubcoreMesh(core_axis_name="c", subcore_axis_name="s")

    @functools.partial(
        pl.kernel,
        mesh=mesh,
        out_type=jax.ShapeDtypeStruct((N, D), jnp.float32),
        compiler_params=pltpu.CompilerParams(use_tc_tiling_on_sc=False),
        scratch_types=[
            pltpu.VMEM((CHUNKS_PER_TILE, PK), jnp.int32),
            pltpu.VMEM((CHUNKS_PER_TILE, PK), jnp.int32),
            pltpu.VMEM((2, NBUF, CHUNK), jnp.int32),
            pltpu.VMEM((2, NBUF, CHUNK), jnp.int32),
            pltpu.VMEM((NBUF, CHUNK, HALF), jnp.float32),
            pltpu.VMEM((NBUF, CHUNK, HALF), jnp.float32),
            pltpu.VMEM_SHARED((N_SH, HALF), jnp.float32),
            pltpu.VMEM_SHARED((N_SH, HALF), jnp.float32),
            pltpu.SemaphoreType.DMA((NBUF,)),
            pltpu.SemaphoreType.DMA((NBUF,)),
            pltpu.SemaphoreType.DMA((NBUF,)),
            pltpu.SemaphoreType.DMA((NBUF,)),
        ],
    )
    def seg_kernel(x_hbm, src_hbm, dst_hbm, out_hbm, spk_v, dpk_v,
                   sidx, didx, rows0, rows1, x_sh, agg_sh,
                   gs0, gs1, ss0, ss1):
        c = lax.axis_index("c")
        s = lax.axis_index("s")
        col0 = c * HALF
        row0 = s * ROWS_PER_TILE
        pltpu.sync_copy(x_hbm.at[pl.ds(row0, ROWS_PER_TILE), pl.ds(col0, HALF)],
                        x_sh.at[pl.ds(row0, ROWS_PER_TILE), :])
        pltpu.sync_copy(x_hbm.at[pl.ds(row0, ROWS_PER_TILE), pl.ds(col0, HALF)],
                        agg_sh.at[pl.ds(row0, ROWS_PER_TILE), :])
        pltpu.sync_copy(src_hbm.at[s], spk_v)
        pltpu.sync_copy(dst_hbm.at[s], dpk_v)
        plsc.subcore_barrier()

        rows = (rows0, rows1)
        gsem = (gs0, gs1)
        ssem = (ss0, ss1)

        def decode(j, p, b):
            for pk_v, out in ((spk_v, sidx), (dpk_v, didx)):
                for m in range(CHUNK // 32):
                    v = pk_v[j, pl.ds(m * 16, 16)]
                    out[p, b, pl.ds(m * 32, 16)] = v & 0xFFFF
                    out[p, b, pl.ds(m * 32 + 16, 16)] = (
                        lax.shift_right_logical(v, 16))

        def start_gather(p, b):
            pltpu.async_copy(x_sh.at[sidx.at[p, b]], rows[p].at[b],
                             gsem[p].at[b])

        def wait_gather(p, b):
            pltpu.make_async_copy(x_sh.at[sidx.at[p, b]], rows[p].at[b],
                                  gsem[p].at[b]).wait()

        def start_scatter(p, b):
            pltpu.async_copy(rows[p].at[b], agg_sh.at[didx.at[p, b]],
                             ssem[p].at[b], add=True)

        def wait_scatter(p, b):
            pltpu.make_async_copy(rows[p].at[b], agg_sh.at[didx.at[p, b]],
                                  ssem[p].at[b]).wait()

        n_dbl = CHUNKS_PER_TILE // (2 * NBUF)
        for p in (0, 1):
            for b in range(NBUF):
                decode(p * NBUF + b, p, b)
                start_gather(p, b)

        def dbl(t, carry):
            base = t * 2 * NBUF
            for p in (0, 1):
                for b in range(NBUF):
                    wait_gather(p, b)
                    start_scatter(p, b)

                @pl.when(t < n_dbl - 1)
                def _():
                    for b in range(NBUF):
                        wait_scatter(p, b)
                        decode(base + p * NBUF + b + 2 * NBUF, p, b)
                        start_gather(p, b)
            return carry

        lax.fori_loop(0, n_dbl, dbl, 0)
        for p in (0, 1):
            for b in range(NBUF):
                wait_scatter(p, b)
        plsc.subcore_barrier()
        pltpu.sync_copy(agg_sh.at[pl.ds(row0, ROWS_PER_TILE), :],
                        out_hbm.at[pl.ds(row0, ROWS_PER_TILE), pl.ds(col0, HALF)])

    return seg_kernel(x, srcpk, dstpk)


def _dense_relu(h, W, b):
    blk = 2000

    def body(h_ref, w_ref, b_ref, o_ref):
        o_ref[...] = jnp.maximum(
            jnp.dot(h_ref[...], w_ref[...],
                    preferred_element_type=jnp.float32) + b_ref[...], 0.0)

    return pl.pallas_call(
        body,
        grid=(N // blk,),
        in_specs=[pl.BlockSpec((blk, D), lambda i: (i, 0)),
                  pl.BlockSpec((D, D), lambda i: (0, 0)),
                  pl.BlockSpec((1, D), lambda i: (0, 0))],
        out_specs=pl.BlockSpec((blk, D), lambda i: (i, 0)),
        out_shape=jax.ShapeDtypeStruct((N, D), jnp.float32),
    )(h, W, b.reshape(1, D))


def _head(h3, x2, wscore, wcls_pad, bcls_pad):
    def body(h_ref, x_ref, ws_ref, wc_ref, bc_ref, o_ref):
        h = h_ref[...]
        s = jnp.tanh(lax.dot_general(h, ws_ref[...], (((1,), (0,)), ((), ())),
                                     preferred_element_type=jnp.float32))
        ii = lax.broadcasted_iota(jnp.int32, (M, M), 0)
        jj = lax.broadcasted_iota(jnp.int32, (M, M), 1)
        eye = (ii == jj).astype(jnp.float32)
        s_row = lax.dot_general(s, eye, (((0,), (0,)), ((), ())),
                                preferred_element_type=jnp.float32)
        s_col_b = lax.broadcast_in_dim(s, (M, M), (0, 1))
        s_row_b = lax.broadcast_in_dim(s_row, (M, M), (0, 1))
        beats = (s_row_b > s_col_b) | ((s_row_b == s_col_b) & (jj < ii))
        rank = jnp.sum(beats.astype(jnp.float32), axis=1, keepdims=True)
        w = jnp.where(rank < float(K), s, 0.0) * (1.0 / K)
        pooled = lax.dot_general(w, x_ref[...], (((0,), (0,)), ((), ())),
                                 preferred_element_type=jnp.float32)
        emb = jnp.maximum(pooled, 0.0)
        logits = jnp.dot(emb, wc_ref[...],
                         preferred_element_type=jnp.float32) + bc_ref[...]
        o_ref[...] = lax.broadcast_in_dim(logits, (8, D), (0, 1))

    out = pl.pallas_call(
        body,
        grid=(B,),
        in_specs=[pl.BlockSpec((M, D), lambda i: (i, 0)),
                  pl.BlockSpec((M, D), lambda i: (i, 0)),
                  pl.BlockSpec((D, 1), lambda i: (0, 0)),
                  pl.BlockSpec((D, D), lambda i: (0, 0)),
                  pl.BlockSpec((1, D), lambda i: (0, 0))],
        out_specs=pl.BlockSpec((8, D), lambda i: (i, 0)),
        out_shape=jax.ShapeDtypeStruct((B * 8, D), jnp.float32),
    )(h3, x2, wscore.reshape(D, 1), wcls_pad, bcls_pad)
    return out[::8]


def kernel(node_feats, collated_edge_index, W1, b1, W2, b2, wscore, Wcls, bcls):
    x = node_feats.reshape(N, D).astype(jnp.float32)
    src = collated_edge_index[0].astype(jnp.int32)
    dst = collated_edge_index[1].astype(jnp.int32)
    srcpk = _pack_idx(src, 0)
    dstpk = _pack_idx(dst, TRASH)

    h1 = _segsum_plus_x(x, srcpk, dstpk)
    x1 = _dense_relu(h1, W1, b1)
    h2 = _segsum_plus_x(x1, srcpk, dstpk)
    x2 = _dense_relu(h2, W2, b2)
    h3 = _segsum_plus_x(x2, srcpk, dstpk)

    wcls_pad = jnp.zeros((D, D), jnp.float32).at[:, :NCLS].set(Wcls)
    bcls_pad = jnp.zeros((1, D), jnp.float32).at[0, :NCLS].set(bcls)
    out = _head(h3, x2, wscore, wcls_pad, bcls_pad)
    return out[:, :NCLS]


# speedup vs baseline: 3.6245x; 1.2366x over previous
import functools

import jax
import jax.numpy as jnp
from jax import lax
from jax.experimental import pallas as pl
from jax.experimental.pallas import tpu as pltpu
from jax.experimental.pallas import tpu_sc as plsc

B = 50
M = 200
D = 128
N = B * M
E = 320000
K = M // 2
NCLS = 11

NC = 2
NS = 16
HALF = D // NC
ROWS_PER_TILE = N // NS
CHUNK = 64
NBUF = 3
CHUNKS_PER_TILE = 324
PK = CHUNK // 2
E_PAD = NS * CHUNKS_PER_TILE * CHUNK
TRASH = N
N_SH = N + 16


def _pack_idx(idx, pad_value):
    half = E_PAD // 2
    lo = idx[:half]
    hi = jnp.pad(idx[half:], (0, E_PAD - E), constant_values=pad_value)
    return (lo | (hi << 16)).reshape(NS, CHUNKS_PER_TILE, PK)


def _segsum_plus_x(x, srcpk, dstpk):
    mesh = plsc.VectorSubcoreMesh(core_axis_name="c", subcore_axis_name="s")

    @functools.partial(
        pl.kernel,
        mesh=mesh,
        out_type=jax.ShapeDtypeStruct((N, D), jnp.float32),
        compiler_params=pltpu.CompilerParams(use_tc_tiling_on_sc=False),
        scratch_types=[
            pltpu.VMEM((CHUNKS_PER_TILE, PK), jnp.int32),
            pltpu.VMEM((CHUNKS_PER_TILE, PK), jnp.int32),
            pltpu.VMEM((2, NBUF, CHUNK), jnp.int32),
            pltpu.VMEM((2, NBUF, CHUNK), jnp.int32),
            pltpu.VMEM((NBUF, CHUNK, HALF), jnp.float32),
            pltpu.VMEM((NBUF, CHUNK, HALF), jnp.float32),
            pltpu.VMEM_SHARED((N_SH, HALF), jnp.float32),
            pltpu.VMEM_SHARED((N_SH, HALF), jnp.float32),
            pltpu.SemaphoreType.DMA((NBUF,)),
            pltpu.SemaphoreType.DMA((NBUF,)),
            pltpu.SemaphoreType.DMA((NBUF,)),
            pltpu.SemaphoreType.DMA((NBUF,)),
        ],
    )
    def seg_kernel(x_hbm, src_hbm, dst_hbm, out_hbm, spk_v, dpk_v,
                   sidx, didx, rows0, rows1, x_sh, agg_sh,
                   gs0, gs1, ss0, ss1):
        c = lax.axis_index("c")
        s = lax.axis_index("s")
        col0 = c * HALF
        row0 = s * ROWS_PER_TILE
        pltpu.sync_copy(x_hbm.at[pl.ds(row0, ROWS_PER_TILE), pl.ds(col0, HALF)],
                        x_sh.at[pl.ds(row0, ROWS_PER_TILE), :])
        pltpu.sync_copy(x_hbm.at[pl.ds(row0, ROWS_PER_TILE), pl.ds(col0, HALF)],
                        agg_sh.at[pl.ds(row0, ROWS_PER_TILE), :])
        pltpu.sync_copy(src_hbm.at[s], spk_v)
        pltpu.sync_copy(dst_hbm.at[s], dpk_v)
        plsc.subcore_barrier()

        rows = (rows0, rows1)
        gsem = (gs0, gs1)
        ssem = (ss0, ss1)

        def decode(j, p, b):
            for pk_v, out in ((spk_v, sidx), (dpk_v, didx)):
                for m in range(CHUNK // 32):
                    v = pk_v[j, pl.ds(m * 16, 16)]
                    out[p, b, pl.ds(m * 32, 16)] = v & 0xFFFF
                    out[p, b, pl.ds(m * 32 + 16, 16)] = (
                        lax.shift_right_logical(v, 16))

        def start_gather(p, b):
            pltpu.async_copy(x_sh.at[sidx.at[p, b]], rows[p].at[b],
                             gsem[p].at[b])

        def wait_gather(p, b):
            pltpu.make_async_copy(x_sh.at[sidx.at[p, b]], rows[p].at[b],
                                  gsem[p].at[b]).wait()

        def start_scatter(p, b):
            pltpu.async_copy(rows[p].at[b], agg_sh.at[didx.at[p, b]],
                             ssem[p].at[b], add=True)

        def wait_scatter(p, b):
            pltpu.make_async_copy(rows[p].at[b], agg_sh.at[didx.at[p, b]],
                                  ssem[p].at[b]).wait()

        n_dbl = CHUNKS_PER_TILE // (2 * NBUF)
        for p in (0, 1):
            for b in range(NBUF):
                decode(p * NBUF + b, p, b)
                start_gather(p, b)

        def dbl(t, carry):
            base = t * 2 * NBUF
            for p in (0, 1):
                for b in range(NBUF):
                    wait_gather(p, b)
                    start_scatter(p, b)

                @pl.when(t < n_dbl - 1)
                def _():
                    for b in range(NBUF):
                        wait_scatter(p, b)
                        decode(base + p * NBUF + b + 2 * NBUF, p, b)
                        start_gather(p, b)
            return carry

        lax.fori_loop(0, n_dbl, dbl, 0)
        for p in (0, 1):
            for b in range(NBUF):
                wait_scatter(p, b)
        plsc.subcore_barrier()
        pltpu.sync_copy(agg_sh.at[pl.ds(row0, ROWS_PER_TILE), :],
                        out_hbm.at[pl.ds(row0, ROWS_PER_TILE), pl.ds(col0, HALF)])

    return seg_kernel(x, srcpk, dstpk)


def _segsum_scalar(t, srcpk, dstpk):
    mesh = plsc.VectorSubcoreMesh(core_axis_name="c", subcore_axis_name="s")
    CPC = CHUNKS_PER_TILE // 2
    RW = 624

    @functools.partial(
        pl.kernel,
        mesh=mesh,
        out_type=jax.ShapeDtypeStruct((NC, N), jnp.float32),
        compiler_params=pltpu.CompilerParams(use_tc_tiling_on_sc=False),
        scratch_types=[
            pltpu.VMEM((CPC, PK), jnp.int32),
            pltpu.VMEM((CPC, PK), jnp.int32),
            pltpu.VMEM((2, NBUF, CHUNK), jnp.int32),
            pltpu.VMEM((2, NBUF, CHUNK), jnp.int32),
            pltpu.VMEM((NBUF, CHUNK), jnp.float32),
            pltpu.VMEM((NBUF, CHUNK), jnp.float32),
            pltpu.VMEM_SHARED((N_SH,), jnp.float32),
            pltpu.VMEM_SHARED((N_SH,), jnp.float32),
            pltpu.SemaphoreType.DMA((NBUF,)),
            pltpu.SemaphoreType.DMA((NBUF,)),
            pltpu.SemaphoreType.DMA((NBUF,)),
            pltpu.SemaphoreType.DMA((NBUF,)),
        ],
    )
    def seg_kernel(t_hbm, src_hbm, dst_hbm, out_hbm, spk_v, dpk_v,
                   sidx, didx, vals0, vals1, t_sh, acc_sh,
                   gs0, gs1, ss0, ss1):
        c = lax.axis_index("c")
        s = lax.axis_index("s")
        r0 = s * RW
        pltpu.sync_copy(t_hbm.at[pl.ds(r0, RW)], t_sh.at[pl.ds(r0, RW)])
        pltpu.sync_copy(t_hbm.at[pl.ds(r0, RW)], acc_sh.at[pl.ds(r0, RW)])

        @pl.when(s == NS - 1)
        def _():
            pltpu.sync_copy(t_hbm.at[pl.ds(RW * NS, N - RW * NS)],
                            t_sh.at[pl.ds(RW * NS, N - RW * NS)])
            pltpu.sync_copy(t_hbm.at[pl.ds(RW * NS, N - RW * NS)],
                            acc_sh.at[pl.ds(RW * NS, N - RW * NS)])

        pltpu.sync_copy(src_hbm.at[s, pl.ds(c * CPC, CPC), :], spk_v)
        pltpu.sync_copy(dst_hbm.at[s, pl.ds(c * CPC, CPC), :], dpk_v)
        plsc.subcore_barrier()

        vals = (vals0, vals1)
        gsem = (gs0, gs1)
        ssem = (ss0, ss1)

        def decode(j, p, b):
            for pk_v, out in ((spk_v, sidx), (dpk_v, didx)):
                for m in range(CHUNK // 32):
                    v = pk_v[j, pl.ds(m * 16, 16)]
                    out[p, b, pl.ds(m * 32, 16)] = v & 0xFFFF
                    out[p, b, pl.ds(m * 32 + 16, 16)] = (
                        lax.shift_right_logical(v, 16))

        def start_gather(p, b):
            pltpu.async_copy(t_sh.at[sidx.at[p, b]], vals[p].at[b],
                             gsem[p].at[b])

        def wait_gather(p, b):
            pltpu.make_async_copy(t_sh.at[sidx.at[p, b]], vals[p].at[b],
                                  gsem[p].at[b]).wait()

        def start_scatter(p, b):
            pltpu.async_copy(vals[p].at[b], acc_sh.at[didx.at[p, b]],
                             ssem[p].at[b], add=True)

        def wait_scatter(p, b):
            pltpu.make_async_copy(vals[p].at[b], acc_sh.at[didx.at[p, b]],
                                  ssem[p].at[b]).wait()

        n_dbl = CPC // (2 * NBUF)
        for p in (0, 1):
            for b in range(NBUF):
                decode(p * NBUF + b, p, b)
                start_gather(p, b)

        def dbl(t_, carry):
            base = t_ * 2 * NBUF
            for p in (0, 1):
                for b in range(NBUF):
                    wait_gather(p, b)
                    start_scatter(p, b)

                @pl.when(t_ < n_dbl - 1)
                def _():
                    for b in range(NBUF):
                        wait_scatter(p, b)
                        decode(base + p * NBUF + b + 2 * NBUF, p, b)
                        start_gather(p, b)
            return carry

        lax.fori_loop(0, n_dbl, dbl, 0)
        for p in (0, 1):
            for b in range(NBUF):
                wait_scatter(p, b)
        plsc.subcore_barrier()
        pltpu.sync_copy(acc_sh.at[pl.ds(r0, RW)], out_hbm.at[c, pl.ds(r0, RW)])

        @pl.when(s == NS - 1)
        def _():
            pltpu.sync_copy(acc_sh.at[pl.ds(RW * NS, N - RW * NS)],
                            out_hbm.at[c, pl.ds(RW * NS, N - RW * NS)])

    return seg_kernel(t, srcpk, dstpk)


def _dense_relu(h, W, b):
    blk = 2000

    def body(h_ref, w_ref, b_ref, o_ref):
        o_ref[...] = jnp.maximum(
            jnp.dot(h_ref[...], w_ref[...],
                    preferred_element_type=jnp.float32) + b_ref[...], 0.0)

    return pl.pallas_call(
        body,
        grid=(N // blk,),
        in_specs=[pl.BlockSpec((blk, D), lambda i: (i, 0)),
                  pl.BlockSpec((D, D), lambda i: (0, 0)),
                  pl.BlockSpec((1, D), lambda i: (0, 0))],
        out_specs=pl.BlockSpec((blk, D), lambda i: (i, 0)),
        out_shape=jax.ShapeDtypeStruct((N, D), jnp.float32),
    )(h, W, b.reshape(1, D))


def _dense_relu_score(h, W, b, wscore):
    blk = 2000

    def body(h_ref, w_ref, b_ref, ws_ref, o_ref, t_ref):
        r = jnp.maximum(
            jnp.dot(h_ref[...], w_ref[...],
                    preferred_element_type=jnp.float32) + b_ref[...], 0.0)
        o_ref[...] = r
        t_ref[...] = lax.dot_general(r, ws_ref[...], (((1,), (0,)), ((), ())),
                                     preferred_element_type=jnp.float32)

    return pl.pallas_call(
        body,
        grid=(N // blk,),
        in_specs=[pl.BlockSpec((blk, D), lambda i: (i, 0)),
                  pl.BlockSpec((D, D), lambda i: (0, 0)),
                  pl.BlockSpec((1, D), lambda i: (0, 0)),
                  pl.BlockSpec((D, 1), lambda i: (0, 0))],
        out_specs=[pl.BlockSpec((blk, D), lambda i: (i, 0)),
                   pl.BlockSpec((blk, 1), lambda i: (i, 0))],
        out_shape=[jax.ShapeDtypeStruct((N, D), jnp.float32),
                   jax.ShapeDtypeStruct((N, 1), jnp.float32)],
    )(h, W, b.reshape(1, D), wscore.reshape(D, 1))


def _head(p0, p1, t, x2, wcls_pad, bcls_pad):
    def body(p0_ref, p1_ref, t_ref, x_ref, wc_ref, bc_ref, o_ref):
        s = jnp.tanh(p0_ref[...] + p1_ref[...] - t_ref[...])
        ii = lax.broadcasted_iota(jnp.int32, (M, M), 0)
        jj = lax.broadcasted_iota(jnp.int32, (M, M), 1)
        eye = (ii == jj).astype(jnp.float32)
        s_row = lax.dot_general(s, eye, (((0,), (0,)), ((), ())),
                                preferred_element_type=jnp.float32)
        s_col_b = lax.broadcast_in_dim(s, (M, M), (0, 1))
        s_row_b = lax.broadcast_in_dim(s_row, (M, M), (0, 1))
        beats = (s_row_b > s_col_b) | ((s_row_b == s_col_b) & (jj < ii))
        rank = jnp.sum(beats.astype(jnp.float32), axis=1, keepdims=True)
        w = jnp.where(rank < float(K), s, 0.0) * (1.0 / K)
        pooled = lax.dot_general(w, x_ref[...], (((0,), (0,)), ((), ())),
                                 preferred_element_type=jnp.float32)
        emb = jnp.maximum(pooled, 0.0)
        logits = jnp.dot(emb, wc_ref[...],
                         preferred_element_type=jnp.float32) + bc_ref[...]
        o_ref[...] = lax.broadcast_in_dim(logits, (8, D), (0, 1))

    out = pl.pallas_call(
        body,
        grid=(B,),
        in_specs=[pl.BlockSpec((M, 1), lambda i: (i, 0)),
                  pl.BlockSpec((M, 1), lambda i: (i, 0)),
                  pl.BlockSpec((M, 1), lambda i: (i, 0)),
                  pl.BlockSpec((M, D), lambda i: (i, 0)),
                  pl.BlockSpec((D, D), lambda i: (0, 0)),
                  pl.BlockSpec((1, D), lambda i: (0, 0))],
        out_specs=pl.BlockSpec((8, D), lambda i: (i, 0)),
        out_shape=jax.ShapeDtypeStruct((B * 8, D), jnp.float32),
    )(p0, p1, t, x2, wcls_pad, bcls_pad)
    return out[::8]


def kernel(node_feats, collated_edge_index, W1, b1, W2, b2, wscore, Wcls, bcls):
    x = node_feats.reshape(N, D).astype(jnp.float32)
    src = collated_edge_index[0].astype(jnp.int32)
    dst = collated_edge_index[1].astype(jnp.int32)
    srcpk = _pack_idx(src, 0)
    dstpk = _pack_idx(dst, TRASH)

    h1 = _segsum_plus_x(x, srcpk, dstpk)
    x1 = _dense_relu(h1, W1, b1)
    h2 = _segsum_plus_x(x1, srcpk, dstpk)
    x2, t = _dense_relu_score(h2, W2, b2, wscore)
    parts = _segsum_scalar(t.reshape(N), srcpk, dstpk)

    wcls_pad = jnp.zeros((D, D), jnp.float32).at[:, :NCLS].set(Wcls)
    bcls_pad = jnp.zeros((1, D), jnp.float32).at[0, :NCLS].set(bcls)
    out = _head(parts[0].reshape(N, 1), parts[1].reshape(N, 1), t, x2,
                wcls_pad, bcls_pad)
    return out[:, :NCLS]


# speedup vs baseline: 3.8691x; 1.0675x over previous
import functools

import jax
import jax.numpy as jnp
from jax import lax
from jax.experimental import pallas as pl
from jax.experimental.pallas import tpu as pltpu
from jax.experimental.pallas import tpu_sc as plsc

B = 50
M = 200
D = 128
N = B * M
E = 320000
K = M // 2
NCLS = 11

NC = 2
NS = 16
HALF = D // NC
ROWS_PER_TILE = N // NS
CHUNK = 64
NBUF = 3
CHUNKS_PER_TILE = 324
PK = CHUNK // 2
E_PAD = NS * CHUNKS_PER_TILE * CHUNK
TRASH = N
N_SH = N + 16


def _pack_idx(idx, pad_value):
    half = E_PAD // 2
    lo = idx[:half]
    hi = jnp.pad(idx[half:], (0, E_PAD - E), constant_values=pad_value)
    return (lo | (hi << 16)).reshape(NS, CHUNKS_PER_TILE, PK)


def _segsum_plus_x(x, srcpk, dstpk):
    mesh = plsc.VectorSubcoreMesh(core_axis_name="c", subcore_axis_name="s")

    @functools.partial(
        pl.kernel,
        mesh=mesh,
        out_type=jax.ShapeDtypeStruct((N, D), jnp.float32),
        compiler_params=pltpu.CompilerParams(use_tc_tiling_on_sc=False),
        scratch_types=[
            pltpu.VMEM((CHUNKS_PER_TILE, PK), jnp.int32),
            pltpu.VMEM((CHUNKS_PER_TILE, PK), jnp.int32),
            pltpu.VMEM((2, NBUF, CHUNK), jnp.int32),
            pltpu.VMEM((2, NBUF, CHUNK), jnp.int32),
            pltpu.VMEM((NBUF, CHUNK, HALF), jnp.float32),
            pltpu.VMEM((NBUF, CHUNK, HALF), jnp.float32),
            pltpu.VMEM_SHARED((N_SH, HALF), jnp.float32),
            pltpu.VMEM_SHARED((N_SH, HALF), jnp.float32),
            pltpu.SemaphoreType.DMA((NBUF,)),
            pltpu.SemaphoreType.DMA((NBUF,)),
            pltpu.SemaphoreType.DMA((NBUF,)),
            pltpu.SemaphoreType.DMA((NBUF,)),
        ],
    )
    def seg_kernel(x_hbm, src_hbm, dst_hbm, out_hbm, spk_v, dpk_v,
                   sidx, didx, rows0, rows1, x_sh, agg_sh,
                   gs0, gs1, ss0, ss1):
        c = lax.axis_index("c")
        s = lax.axis_index("s")
        col0 = c * HALF
        row0 = s * ROWS_PER_TILE
        pltpu.sync_copy(x_hbm.at[pl.ds(row0, ROWS_PER_TILE), pl.ds(col0, HALF)],
                        x_sh.at[pl.ds(row0, ROWS_PER_TILE), :])
        pltpu.sync_copy(x_hbm.at[pl.ds(row0, ROWS_PER_TILE), pl.ds(col0, HALF)],
                        agg_sh.at[pl.ds(row0, ROWS_PER_TILE), :])
        pltpu.sync_copy(src_hbm.at[s], spk_v)
        pltpu.sync_copy(dst_hbm.at[s], dpk_v)
        plsc.subcore_barrier()

        rows = (rows0, rows1)
        gsem = (gs0, gs1)
        ssem = (ss0, ss1)

        def decode(j, p, b):
            for pk_v, out in ((spk_v, sidx), (dpk_v, didx)):
                for m in range(CHUNK // 32):
                    v = pk_v[j, pl.ds(m * 16, 16)]
                    out[p, b, pl.ds(m * 32, 16)] = v & 0xFFFF
                    out[p, b, pl.ds(m * 32 + 16, 16)] = (
                        lax.shift_right_logical(v, 16))

        def start_gather(p, b):
            pltpu.async_copy(x_sh.at[sidx.at[p, b]], rows[p].at[b],
                             gsem[p].at[b])

        def wait_gather(p, b):
            pltpu.make_async_copy(x_sh.at[sidx.at[p, b]], rows[p].at[b],
                                  gsem[p].at[b]).wait()

        def start_scatter(p, b):
            pltpu.async_copy(rows[p].at[b], agg_sh.at[didx.at[p, b]],
                             ssem[p].at[b], add=True)

        def wait_scatter(p, b):
            pltpu.make_async_copy(rows[p].at[b], agg_sh.at[didx.at[p, b]],
                                  ssem[p].at[b]).wait()

        n_dbl = CHUNKS_PER_TILE // (2 * NBUF)
        for p in (0, 1):
            for b in range(NBUF):
                decode(p * NBUF + b, p, b)
                start_gather(p, b)

        def dbl(t, carry):
            base = t * 2 * NBUF
            for p in (0, 1):
                for b in range(NBUF):
                    wait_gather(p, b)
                    start_scatter(p, b)

                @pl.when(t < n_dbl - 1)
                def _():
                    for b in range(NBUF):
                        wait_scatter(p, b)
                        decode(base + p * NBUF + b + 2 * NBUF, p, b)
                        start_gather(p, b)
            return carry

        lax.fori_loop(0, n_dbl, dbl, 0)
        for p in (0, 1):
            for b in range(NBUF):
                wait_scatter(p, b)
        plsc.subcore_barrier()
        pltpu.sync_copy(agg_sh.at[pl.ds(row0, ROWS_PER_TILE), :],
                        out_hbm.at[pl.ds(row0, ROWS_PER_TILE), pl.ds(col0, HALF)])

    return seg_kernel(x, srcpk, dstpk)


def _segsum_scalar(t, srcpk, dstpk):
    mesh = plsc.VectorSubcoreMesh(core_axis_name="c", subcore_axis_name="s")
    CPC = CHUNKS_PER_TILE // 2
    RW = 624

    @functools.partial(
        pl.kernel,
        mesh=mesh,
        out_type=jax.ShapeDtypeStruct((NC, N), jnp.float32),
        compiler_params=pltpu.CompilerParams(use_tc_tiling_on_sc=False),
        scratch_types=[
            pltpu.VMEM((CPC, PK), jnp.int32),
            pltpu.VMEM((CPC, PK), jnp.int32),
            pltpu.VMEM((2, NBUF, CHUNK), jnp.int32),
            pltpu.VMEM((2, NBUF, CHUNK), jnp.int32),
            pltpu.VMEM((NBUF, CHUNK), jnp.float32),
            pltpu.VMEM((NBUF, CHUNK), jnp.float32),
            pltpu.VMEM_SHARED((N_SH,), jnp.float32),
            pltpu.VMEM_SHARED((N_SH,), jnp.float32),
            pltpu.SemaphoreType.DMA((NBUF,)),
            pltpu.SemaphoreType.DMA((NBUF,)),
            pltpu.SemaphoreType.DMA((NBUF,)),
            pltpu.SemaphoreType.DMA((NBUF,)),
        ],
    )
    def seg_kernel(t_hbm, src_hbm, dst_hbm, out_hbm, spk_v, dpk_v,
                   sidx, didx, vals0, vals1, t_sh, acc_sh,
                   gs0, gs1, ss0, ss1):
        c = lax.axis_index("c")
        s = lax.axis_index("s")
        r0 = s * RW
        pltpu.sync_copy(t_hbm.at[pl.ds(r0, RW)], t_sh.at[pl.ds(r0, RW)])
        pltpu.sync_copy(t_hbm.at[pl.ds(r0, RW)], acc_sh.at[pl.ds(r0, RW)])

        @pl.when(s == NS - 1)
        def _():
            pltpu.sync_copy(t_hbm.at[pl.ds(RW * NS, N - RW * NS)],
                            t_sh.at[pl.ds(RW * NS, N - RW * NS)])
            pltpu.sync_copy(t_hbm.at[pl.ds(RW * NS, N - RW * NS)],
                            acc_sh.at[pl.ds(RW * NS, N - RW * NS)])

        pltpu.sync_copy(src_hbm.at[s, pl.ds(c * CPC, CPC), :], spk_v)
        pltpu.sync_copy(dst_hbm.at[s, pl.ds(c * CPC, CPC), :], dpk_v)
        plsc.subcore_barrier()

        vals = (vals0, vals1)
        gsem = (gs0, gs1)
        ssem = (ss0, ss1)

        def decode(j, p, b):
            for pk_v, out in ((spk_v, sidx), (dpk_v, didx)):
                for m in range(CHUNK // 32):
                    v = pk_v[j, pl.ds(m * 16, 16)]
                    out[p, b, pl.ds(m * 32, 16)] = v & 0xFFFF
                    out[p, b, pl.ds(m * 32 + 16, 16)] = (
                        lax.shift_right_logical(v, 16))

        def start_gather(p, b):
            pltpu.async_copy(t_sh.at[sidx.at[p, b]], vals[p].at[b],
                             gsem[p].at[b])

        def wait_gather(p, b):
            pltpu.make_async_copy(t_sh.at[sidx.at[p, b]], vals[p].at[b],
                                  gsem[p].at[b]).wait()

        def start_scatter(p, b):
            pltpu.async_copy(vals[p].at[b], acc_sh.at[didx.at[p, b]],
                             ssem[p].at[b], add=True)

        def wait_scatter(p, b):
            pltpu.make_async_copy(vals[p].at[b], acc_sh.at[didx.at[p, b]],
                                  ssem[p].at[b]).wait()

        n_dbl = CPC // (2 * NBUF)
        for p in (0, 1):
            for b in range(NBUF):
                decode(p * NBUF + b, p, b)
                start_gather(p, b)

        def dbl(t_, carry):
            base = t_ * 2 * NBUF
            for p in (0, 1):
                for b in range(NBUF):
                    wait_gather(p, b)
                    start_scatter(p, b)

                @pl.when(t_ < n_dbl - 1)
                def _():
                    for b in range(NBUF):
                        wait_scatter(p, b)
                        decode(base + p * NBUF + b + 2 * NBUF, p, b)
                        start_gather(p, b)
            return carry

        lax.fori_loop(0, n_dbl, dbl, 0)
        for p in (0, 1):
            for b in range(NBUF):
                wait_scatter(p, b)
        plsc.subcore_barrier()
        pltpu.sync_copy(acc_sh.at[pl.ds(r0, RW)], out_hbm.at[c, pl.ds(r0, RW)])

        @pl.when(s == NS - 1)
        def _():
            pltpu.sync_copy(acc_sh.at[pl.ds(RW * NS, N - RW * NS)],
                            out_hbm.at[c, pl.ds(RW * NS, N - RW * NS)])

    return seg_kernel(t, srcpk, dstpk)


def _dense_relu(h, W, b):
    blk = 2000

    def body(h_ref, w_ref, b_ref, o_ref):
        o_ref[...] = jnp.maximum(
            jnp.dot(h_ref[...], w_ref[...],
                    preferred_element_type=jnp.float32) + b_ref[...], 0.0)

    return pl.pallas_call(
        body,
        grid=(N // blk,),
        in_specs=[pl.BlockSpec((blk, D), lambda i: (i, 0)),
                  pl.BlockSpec((D, D), lambda i: (0, 0)),
                  pl.BlockSpec((1, D), lambda i: (0, 0))],
        out_specs=pl.BlockSpec((blk, D), lambda i: (i, 0)),
        out_shape=jax.ShapeDtypeStruct((N, D), jnp.float32),
    )(h, W, b.reshape(1, D))


def _dense_relu_score(h, W, b, wscore):
    blk = 2000

    def body(h_ref, w_ref, b_ref, ws_ref, o_ref, t_ref):
        r = jnp.maximum(
            jnp.dot(h_ref[...], w_ref[...],
                    preferred_element_type=jnp.float32) + b_ref[...], 0.0)
        o_ref[...] = r
        t_ref[...] = lax.dot_general(r, ws_ref[...], (((1,), (0,)), ((), ())),
                                     preferred_element_type=jnp.float32)

    return pl.pallas_call(
        body,
        grid=(N // blk,),
        in_specs=[pl.BlockSpec((blk, D), lambda i: (i, 0)),
                  pl.BlockSpec((D, D), lambda i: (0, 0)),
                  pl.BlockSpec((1, D), lambda i: (0, 0)),
                  pl.BlockSpec((D, 1), lambda i: (0, 0))],
        out_specs=[pl.BlockSpec((blk, D), lambda i: (i, 0)),
                   pl.BlockSpec((blk, 1), lambda i: (i, 0))],
        out_shape=[jax.ShapeDtypeStruct((N, D), jnp.float32),
                   jax.ShapeDtypeStruct((N, 1), jnp.float32)],
    )(h, W, b.reshape(1, D), wscore.reshape(D, 1))


def _head(parts, t, x2, wcls_pad, bcls_pad):
    GB = 10
    R = GB * M
    pc = parts.reshape(2 * N, 1)
    pr = parts.reshape(2 * B, M)
    tc_ = t
    tr = t.reshape(B, M)

    def body(p0c, p1c, pr_ref, tc_ref, tr_ref, x_ref, wc_ref, bc_ref, o_ref):
        pid = pl.program_id(0)
        s_col = jnp.tanh(p0c[...] + p1c[...] - tc_ref[...])
        sg = jnp.tanh(pr_ref[0] + pr_ref[1] - tr_ref[...])
        ii = lax.broadcasted_iota(jnp.int32, (R, B), 0) + pid * R
        gg = lax.broadcasted_iota(jnp.int32, (R, B), 1)
        P = ((ii >= gg * M) & (ii < gg * M + M)).astype(jnp.float32)
        B2 = lax.dot_general(P, sg, (((1,), (0,)), ((), ())),
                             preferred_element_type=jnp.float32)
        B1 = lax.broadcast_in_dim(s_col, (R, M), (0, 1))
        ids = lax.broadcasted_iota(jnp.int32, (B, 1), 0).astype(jnp.float32)
        gid = lax.dot_general(P, ids, (((1,), (0,)), ((), ())),
                              preferred_element_type=jnp.float32)
        icol = (lax.broadcasted_iota(jnp.int32, (R, 1), 0) + pid * R
                ).astype(jnp.float32)
        im = icol - gid * M
        imb = lax.broadcast_in_dim(im, (R, M), (0, 1))
        jj = lax.broadcasted_iota(jnp.int32, (R, M), 1).astype(jnp.float32)
        beats = (B2 > B1) | ((B2 == B1) & (jj < imb))
        rank = jnp.sum(beats.astype(jnp.float32), axis=1, keepdims=True)
        w = jnp.where(rank < float(K), s_col, 0.0) * (1.0 / K)
        wx = w * x_ref[...]
        pooled = lax.dot_general(P, wx, (((0,), (0,)), ((), ())),
                                 preferred_element_type=jnp.float32)
        emb = jnp.maximum(pooled, 0.0)
        logits = jnp.dot(emb, wc_ref[...],
                         preferred_element_type=jnp.float32) + bc_ref[...]
        rr = lax.broadcasted_iota(jnp.int32, (8 * GB, B), 0)
        gg2 = (lax.broadcasted_iota(jnp.int32, (8 * GB, B), 1)
               - pid * GB) * 8
        Q = ((rr >= gg2) & (rr < gg2 + 8)).astype(jnp.float32)
        o_ref[...] = lax.dot_general(Q, logits, (((1,), (0,)), ((), ())),
                                     preferred_element_type=jnp.float32)

    out = pl.pallas_call(
        body,
        grid=(B // GB,),
        in_specs=[pl.BlockSpec((R, 1), lambda i: (i, 0)),
                  pl.BlockSpec((R, 1), lambda i: (i + B // GB, 0)),
                  pl.BlockSpec((2, B, M), lambda i: (0, 0, 0)),
                  pl.BlockSpec((R, 1), lambda i: (i, 0)),
                  pl.BlockSpec((B, M), lambda i: (0, 0)),
                  pl.BlockSpec((R, D), lambda i: (i, 0)),
                  pl.BlockSpec((D, D), lambda i: (0, 0)),
                  pl.BlockSpec((1, D), lambda i: (0, 0))],
        out_specs=pl.BlockSpec((8 * GB, D), lambda i: (i, 0)),
        out_shape=jax.ShapeDtypeStruct((B * 8, D), jnp.float32),
    )(pc, pc, pr.reshape(2, B, M), tc_, tr, x2, wcls_pad, bcls_pad)
    return out[::8]


def kernel(node_feats, collated_edge_index, W1, b1, W2, b2, wscore, Wcls, bcls):
    x = node_feats.reshape(N, D).astype(jnp.float32)
    src = collated_edge_index[0].astype(jnp.int32)
    dst = collated_edge_index[1].astype(jnp.int32)
    srcpk = _pack_idx(src, 0)
    dstpk = _pack_idx(dst, TRASH)

    h1 = _segsum_plus_x(x, srcpk, dstpk)
    x1 = _dense_relu(h1, W1, b1)
    h2 = _segsum_plus_x(x1, srcpk, dstpk)
    x2, t = _dense_relu_score(h2, W2, b2, wscore)
    parts = _segsum_scalar(t.reshape(N), srcpk, dstpk)

    wcls_pad = jnp.zeros((D, D), jnp.float32).at[:, :NCLS].set(Wcls)
    bcls_pad = jnp.zeros((1, D), jnp.float32).at[0, :NCLS].set(bcls)
    out = _head(parts, t, x2, wcls_pad, bcls_pad)
    return out[:, :NCLS]


# speedup vs baseline: 4.0361x; 1.0432x over previous
import functools

import jax
import jax.numpy as jnp
from jax import lax
from jax.experimental import pallas as pl
from jax.experimental.pallas import tpu as pltpu
from jax.experimental.pallas import tpu_sc as plsc

B = 50
M = 200
D = 128
N = B * M
E = 320000
K = M // 2
NCLS = 11

NC = 2
NS = 16
HALF = D // NC
ROWS_PER_TILE = N // NS
CHUNK = 32
NBUF = 6
CHUNKS_PER_TILE = 648
PK = CHUNK // 2
E_PAD = NS * CHUNKS_PER_TILE * CHUNK
TRASH = N
N_SH = N + 16


def _pack_idx(idx, pad_value):
    half = E_PAD // 2
    lo = idx[:half]
    hi = jnp.pad(idx[half:], (0, E_PAD - E), constant_values=pad_value)
    return (lo | (hi << 16)).reshape(NS, CHUNKS_PER_TILE, PK)


def _segsum_plus_x(x, srcpk, dstpk):
    mesh = plsc.VectorSubcoreMesh(core_axis_name="c", subcore_axis_name="s")

    @functools.partial(
        pl.kernel,
        mesh=mesh,
        out_type=jax.ShapeDtypeStruct((N, D), jnp.float32),
        compiler_params=pltpu.CompilerParams(use_tc_tiling_on_sc=False),
        scratch_types=[
            pltpu.VMEM((CHUNKS_PER_TILE, PK), jnp.int32),
            pltpu.VMEM((CHUNKS_PER_TILE, PK), jnp.int32),
            pltpu.VMEM((2, NBUF, CHUNK), jnp.int32),
            pltpu.VMEM((2, NBUF, CHUNK), jnp.int32),
            pltpu.VMEM((NBUF, CHUNK, HALF), jnp.float32),
            pltpu.VMEM((NBUF, CHUNK, HALF), jnp.float32),
            pltpu.VMEM_SHARED((N_SH, HALF), jnp.float32),
            pltpu.VMEM_SHARED((N_SH, HALF), jnp.float32),
            pltpu.SemaphoreType.DMA((NBUF,)),
            pltpu.SemaphoreType.DMA((NBUF,)),
            pltpu.SemaphoreType.DMA((NBUF,)),
            pltpu.SemaphoreType.DMA((NBUF,)),
        ],
    )
    def seg_kernel(x_hbm, src_hbm, dst_hbm, out_hbm, spk_v, dpk_v,
                   sidx, didx, rows0, rows1, x_sh, agg_sh,
                   gs0, gs1, ss0, ss1):
        c = lax.axis_index("c")
        s = lax.axis_index("s")
        col0 = c * HALF
        row0 = s * ROWS_PER_TILE
        pltpu.sync_copy(x_hbm.at[pl.ds(row0, ROWS_PER_TILE), pl.ds(col0, HALF)],
                        x_sh.at[pl.ds(row0, ROWS_PER_TILE), :])
        pltpu.sync_copy(x_hbm.at[pl.ds(row0, ROWS_PER_TILE), pl.ds(col0, HALF)],
                        agg_sh.at[pl.ds(row0, ROWS_PER_TILE), :])
        pltpu.sync_copy(src_hbm.at[s], spk_v)
        pltpu.sync_copy(dst_hbm.at[s], dpk_v)
        plsc.subcore_barrier()

        rows = (rows0, rows1)
        gsem = (gs0, gs1)
        ssem = (ss0, ss1)

        def decode(j, p, b):
            for pk_v, out in ((spk_v, sidx), (dpk_v, didx)):
                for m in range(CHUNK // 32):
                    v = pk_v[j, pl.ds(m * 16, 16)]
                    out[p, b, pl.ds(m * 32, 16)] = v & 0xFFFF
                    out[p, b, pl.ds(m * 32 + 16, 16)] = (
                        lax.shift_right_logical(v, 16))

        def start_gather(p, b):
            pltpu.async_copy(x_sh.at[sidx.at[p, b]], rows[p].at[b],
                             gsem[p].at[b])

        def wait_gather(p, b):
            pltpu.make_async_copy(x_sh.at[sidx.at[p, b]], rows[p].at[b],
                                  gsem[p].at[b]).wait()

        def start_scatter(p, b):
            pltpu.async_copy(rows[p].at[b], agg_sh.at[didx.at[p, b]],
                             ssem[p].at[b], add=True)

        def wait_scatter(p, b):
            pltpu.make_async_copy(rows[p].at[b], agg_sh.at[didx.at[p, b]],
                                  ssem[p].at[b]).wait()

        n_dbl = CHUNKS_PER_TILE // (2 * NBUF)
        for p in (0, 1):
            for b in range(NBUF):
                decode(p * NBUF + b, p, b)
                start_gather(p, b)

        def dbl(t, carry):
            base = t * 2 * NBUF
            for p in (0, 1):
                for b in range(NBUF):
                    wait_gather(p, b)
                    start_scatter(p, b)

                @pl.when(t < n_dbl - 1)
                def _():
                    for b in range(NBUF):
                        wait_scatter(p, b)
                        decode(base + p * NBUF + b + 2 * NBUF, p, b)
                        start_gather(p, b)
            return carry

        lax.fori_loop(0, n_dbl, dbl, 0)
        for p in (0, 1):
            for b in range(NBUF):
                wait_scatter(p, b)
        plsc.subcore_barrier()
        pltpu.sync_copy(agg_sh.at[pl.ds(row0, ROWS_PER_TILE), :],
                        out_hbm.at[pl.ds(row0, ROWS_PER_TILE), pl.ds(col0, HALF)])

    return seg_kernel(x, srcpk, dstpk)


def _segsum_scalar(t, srcpk, dstpk):
    mesh = plsc.VectorSubcoreMesh(core_axis_name="c", subcore_axis_name="s")
    CPC = CHUNKS_PER_TILE // 2
    RW = 624

    @functools.partial(
        pl.kernel,
        mesh=mesh,
        out_type=jax.ShapeDtypeStruct((NC, N), jnp.float32),
        compiler_params=pltpu.CompilerParams(use_tc_tiling_on_sc=False),
        scratch_types=[
            pltpu.VMEM((CPC, PK), jnp.int32),
            pltpu.VMEM((CPC, PK), jnp.int32),
            pltpu.VMEM((2, NBUF, CHUNK), jnp.int32),
            pltpu.VMEM((2, NBUF, CHUNK), jnp.int32),
            pltpu.VMEM((NBUF, CHUNK), jnp.float32),
            pltpu.VMEM((NBUF, CHUNK), jnp.float32),
            pltpu.VMEM_SHARED((N_SH,), jnp.float32),
            pltpu.VMEM_SHARED((N_SH,), jnp.float32),
            pltpu.SemaphoreType.DMA((NBUF,)),
            pltpu.SemaphoreType.DMA((NBUF,)),
            pltpu.SemaphoreType.DMA((NBUF,)),
            pltpu.SemaphoreType.DMA((NBUF,)),
        ],
    )
    def seg_kernel(t_hbm, src_hbm, dst_hbm, out_hbm, spk_v, dpk_v,
                   sidx, didx, vals0, vals1, t_sh, acc_sh,
                   gs0, gs1, ss0, ss1):
        c = lax.axis_index("c")
        s = lax.axis_index("s")
        r0 = s * RW
        pltpu.sync_copy(t_hbm.at[pl.ds(r0, RW)], t_sh.at[pl.ds(r0, RW)])
        pltpu.sync_copy(t_hbm.at[pl.ds(r0, RW)], acc_sh.at[pl.ds(r0, RW)])

        @pl.when(s == NS - 1)
        def _():
            pltpu.sync_copy(t_hbm.at[pl.ds(RW * NS, N - RW * NS)],
                            t_sh.at[pl.ds(RW * NS, N - RW * NS)])
            pltpu.sync_copy(t_hbm.at[pl.ds(RW * NS, N - RW * NS)],
                            acc_sh.at[pl.ds(RW * NS, N - RW * NS)])

        pltpu.sync_copy(src_hbm.at[s, pl.ds(c * CPC, CPC), :], spk_v)
        pltpu.sync_copy(dst_hbm.at[s, pl.ds(c * CPC, CPC), :], dpk_v)
        plsc.subcore_barrier()

        vals = (vals0, vals1)
        gsem = (gs0, gs1)
        ssem = (ss0, ss1)

        def decode(j, p, b):
            for pk_v, out in ((spk_v, sidx), (dpk_v, didx)):
                for m in range(CHUNK // 32):
                    v = pk_v[j, pl.ds(m * 16, 16)]
                    out[p, b, pl.ds(m * 32, 16)] = v & 0xFFFF
                    out[p, b, pl.ds(m * 32 + 16, 16)] = (
                        lax.shift_right_logical(v, 16))

        def start_gather(p, b):
            pltpu.async_copy(t_sh.at[sidx.at[p, b]], vals[p].at[b],
                             gsem[p].at[b])

        def wait_gather(p, b):
            pltpu.make_async_copy(t_sh.at[sidx.at[p, b]], vals[p].at[b],
                                  gsem[p].at[b]).wait()

        def start_scatter(p, b):
            pltpu.async_copy(vals[p].at[b], acc_sh.at[didx.at[p, b]],
                             ssem[p].at[b], add=True)

        def wait_scatter(p, b):
            pltpu.make_async_copy(vals[p].at[b], acc_sh.at[didx.at[p, b]],
                                  ssem[p].at[b]).wait()

        n_dbl = CPC // (2 * NBUF)
        for p in (0, 1):
            for b in range(NBUF):
                decode(p * NBUF + b, p, b)
                start_gather(p, b)

        def dbl(t_, carry):
            base = t_ * 2 * NBUF
            for p in (0, 1):
                for b in range(NBUF):
                    wait_gather(p, b)
                    start_scatter(p, b)

                @pl.when(t_ < n_dbl - 1)
                def _():
                    for b in range(NBUF):
                        wait_scatter(p, b)
                        decode(base + p * NBUF + b + 2 * NBUF, p, b)
                        start_gather(p, b)
            return carry

        lax.fori_loop(0, n_dbl, dbl, 0)
        for p in (0, 1):
            for b in range(NBUF):
                wait_scatter(p, b)
        plsc.subcore_barrier()
        pltpu.sync_copy(acc_sh.at[pl.ds(r0, RW)], out_hbm.at[c, pl.ds(r0, RW)])

        @pl.when(s == NS - 1)
        def _():
            pltpu.sync_copy(acc_sh.at[pl.ds(RW * NS, N - RW * NS)],
                            out_hbm.at[c, pl.ds(RW * NS, N - RW * NS)])

    return seg_kernel(t, srcpk, dstpk)


def _dense_relu(h, W, b):
    blk = 2000

    def body(h_ref, w_ref, b_ref, o_ref):
        o_ref[...] = jnp.maximum(
            jnp.dot(h_ref[...], w_ref[...],
                    preferred_element_type=jnp.float32) + b_ref[...], 0.0)

    return pl.pallas_call(
        body,
        grid=(N // blk,),
        in_specs=[pl.BlockSpec((blk, D), lambda i: (i, 0)),
                  pl.BlockSpec((D, D), lambda i: (0, 0)),
                  pl.BlockSpec((1, D), lambda i: (0, 0))],
        out_specs=pl.BlockSpec((blk, D), lambda i: (i, 0)),
        out_shape=jax.ShapeDtypeStruct((N, D), jnp.float32),
    )(h, W, b.reshape(1, D))


def _dense_relu_score(h, W, b, wscore):
    blk = 2000

    def body(h_ref, w_ref, b_ref, ws_ref, o_ref, t_ref):
        r = jnp.maximum(
            jnp.dot(h_ref[...], w_ref[...],
                    preferred_element_type=jnp.float32) + b_ref[...], 0.0)
        o_ref[...] = r
        t_ref[...] = lax.dot_general(r, ws_ref[...], (((1,), (0,)), ((), ())),
                                     preferred_element_type=jnp.float32)

    return pl.pallas_call(
        body,
        grid=(N // blk,),
        in_specs=[pl.BlockSpec((blk, D), lambda i: (i, 0)),
                  pl.BlockSpec((D, D), lambda i: (0, 0)),
                  pl.BlockSpec((1, D), lambda i: (0, 0)),
                  pl.BlockSpec((D, 1), lambda i: (0, 0))],
        out_specs=[pl.BlockSpec((blk, D), lambda i: (i, 0)),
                   pl.BlockSpec((blk, 1), lambda i: (i, 0))],
        out_shape=[jax.ShapeDtypeStruct((N, D), jnp.float32),
                   jax.ShapeDtypeStruct((N, 1), jnp.float32)],
    )(h, W, b.reshape(1, D), wscore.reshape(D, 1))


def _head(parts, t, x2, wcls_pad, bcls_pad):
    GB = 10
    R = GB * M
    pc = parts.reshape(2 * N, 1)
    pr = parts.reshape(2 * B, M)
    tc_ = t
    tr = t.reshape(B, M)

    def body(p0c, p1c, pr_ref, tc_ref, tr_ref, x_ref, wc_ref, bc_ref, o_ref):
        pid = pl.program_id(0)
        s_col = jnp.tanh(p0c[...] + p1c[...] - tc_ref[...])
        sg = jnp.tanh(pr_ref[0] + pr_ref[1] - tr_ref[...])
        ii = lax.broadcasted_iota(jnp.int32, (R, B), 0) + pid * R
        gg = lax.broadcasted_iota(jnp.int32, (R, B), 1)
        P = ((ii >= gg * M) & (ii < gg * M + M)).astype(jnp.float32)
        B2 = lax.dot_general(P, sg, (((1,), (0,)), ((), ())),
                             preferred_element_type=jnp.float32)
        B1 = lax.broadcast_in_dim(s_col, (R, M), (0, 1))
        ids = lax.broadcasted_iota(jnp.int32, (B, 1), 0).astype(jnp.float32)
        gid = lax.dot_general(P, ids, (((1,), (0,)), ((), ())),
                              preferred_element_type=jnp.float32)
        icol = (lax.broadcasted_iota(jnp.int32, (R, 1), 0) + pid * R
                ).astype(jnp.float32)
        im = icol - gid * M
        imb = lax.broadcast_in_dim(im, (R, M), (0, 1))
        jj = lax.broadcasted_iota(jnp.int32, (R, M), 1).astype(jnp.float32)
        beats = (B2 > B1) | ((B2 == B1) & (jj < imb))
        rank = jnp.sum(beats.astype(jnp.float32), axis=1, keepdims=True)
        w = jnp.where(rank < float(K), s_col, 0.0) * (1.0 / K)
        wx = w * x_ref[...]
        pooled = lax.dot_general(P, wx, (((0,), (0,)), ((), ())),
                                 preferred_element_type=jnp.float32)
        emb = jnp.maximum(pooled, 0.0)
        logits = jnp.dot(emb, wc_ref[...],
                         preferred_element_type=jnp.float32) + bc_ref[...]
        rr = lax.broadcasted_iota(jnp.int32, (8 * GB, B), 0)
        gg2 = (lax.broadcasted_iota(jnp.int32, (8 * GB, B), 1)
               - pid * GB) * 8
        Q = ((rr >= gg2) & (rr < gg2 + 8)).astype(jnp.float32)
        o_ref[...] = lax.dot_general(Q, logits, (((1,), (0,)), ((), ())),
                                     preferred_element_type=jnp.float32)

    out = pl.pallas_call(
        body,
        grid=(B // GB,),
        in_specs=[pl.BlockSpec((R, 1), lambda i: (i, 0)),
                  pl.BlockSpec((R, 1), lambda i: (i + B // GB, 0)),
                  pl.BlockSpec((2, B, M), lambda i: (0, 0, 0)),
                  pl.BlockSpec((R, 1), lambda i: (i, 0)),
                  pl.BlockSpec((B, M), lambda i: (0, 0)),
                  pl.BlockSpec((R, D), lambda i: (i, 0)),
                  pl.BlockSpec((D, D), lambda i: (0, 0)),
                  pl.BlockSpec((1, D), lambda i: (0, 0))],
        out_specs=pl.BlockSpec((8 * GB, D), lambda i: (i, 0)),
        out_shape=jax.ShapeDtypeStruct((B * 8, D), jnp.float32),
    )(pc, pc, pr.reshape(2, B, M), tc_, tr, x2, wcls_pad, bcls_pad)
    return out[::8]


def kernel(node_feats, collated_edge_index, W1, b1, W2, b2, wscore, Wcls, bcls):
    x = node_feats.reshape(N, D).astype(jnp.float32)
    src = collated_edge_index[0].astype(jnp.int32)
    dst = collated_edge_index[1].astype(jnp.int32)
    srcpk = _pack_idx(src, 0)
    dstpk = _pack_idx(dst, TRASH)

    h1 = _segsum_plus_x(x, srcpk, dstpk)
    x1 = _dense_relu(h1, W1, b1)
    h2 = _segsum_plus_x(x1, srcpk, dstpk)
    x2, t = _dense_relu_score(h2, W2, b2, wscore)
    parts = _segsum_scalar(t.reshape(N), srcpk, dstpk)

    wcls_pad = jnp.zeros((D, D), jnp.float32).at[:, :NCLS].set(Wcls)
    bcls_pad = jnp.zeros((1, D), jnp.float32).at[0, :NCLS].set(bcls)
    out = _head(parts, t, x2, wcls_pad, bcls_pad)
    return out[:, :NCLS]
